# baseline (speedup 1.0000x reference)
import jax
import jax.numpy as jnp
from jax.experimental import pallas as pl
from jax.experimental.pallas import tpu as pltpu

N_P = 8
N_S = 64
N_A = N_P + N_S
COMM_RANGE = 0.3
HID = 256
B = 32
HB = B // 2

_EXACT = jax.lax.Precision.HIGHEST


def _iota2(shape, dim):
    return jax.lax.broadcasted_iota(jnp.int32, shape, dim)


def _body(obs_ref, ph_ref, s_hbm_ref, wq_ref, wk_ref, wv_ref, fcw_ref,
          fcb_ref, h_out_ref, alpha_ref, s0_ref, s1_ref, sem0, sem1):
    R = B * N_P
    C = B * N_S
    HC = HB * N_S

    cp0 = pltpu.make_async_copy(s_hbm_ref.at[0:HB], s0_ref, sem0)
    cp1 = pltpu.make_async_copy(s_hbm_ref.at[HB:B], s1_ref, sem1)
    cp0.start()
    cp1.start()

    ph = ph_ref[...]

    ob = obs_ref[...]
    pxl = ob[:, 0:1, :N_P].reshape(B, N_P)
    pyl = ob[:, 1:2, :N_P].reshape(B, N_P)
    sxl = ob[:, 0:1, N_P:].reshape(B, N_S)
    syl = ob[:, 1:2, N_P:].reshape(B, N_S)
    xy = jnp.concatenate([pxl, pyl, sxl, syl], axis=1)
    eb = (_iota2((R, B), 0) // N_P == _iota2((R, B), 1)).astype(jnp.float32)
    xyr = jnp.dot(eb, xy, precision=_EXACT)
    sx = xyr[:, 2 * N_P:2 * N_P + N_S]
    sy = xyr[:, 2 * N_P + N_S:]
    lane_eq = _iota2((R, N_P), 1) == _iota2((R, N_P), 0) % N_P
    px = jnp.sum(jnp.where(lane_eq, xyr[:, :N_P], 0.0),
                 axis=1, keepdims=True)
    py = jnp.sum(jnp.where(lane_eq, xyr[:, N_P:2 * N_P], 0.0),
                 axis=1, keepdims=True)

    near2 = jnp.maximum(jnp.abs(px - sx), jnp.abs(py - sy)) <= COMM_RANGE
    has_vis = jnp.any(near2, axis=1, keepdims=True)
    near2f = jnp.where(near2, 1.0, 0.0)
    mask = ((jnp.concatenate([near2f] * B, axis=1) > 0.5) &
            (_iota2((R, C), 0) // N_P == _iota2((R, C), 1) // N_S))

    qs = (jnp.dot(ph.astype(jnp.bfloat16), wq_ref[...].astype(jnp.bfloat16),
                  preferred_element_type=jnp.float32)
          * (1.0 / 16.0)).astype(jnp.bfloat16)
    wk = wk_ref[...].astype(jnp.bfloat16)
    wv = wv_ref[...].astype(jnp.bfloat16)

    cp0.wait()
    s0 = s0_ref[...].reshape(HB * N_S, HID).astype(jnp.bfloat16)
    k0 = jnp.dot(s0, wk, preferred_element_type=jnp.float32)
    v0 = jnp.dot(s0, wv, preferred_element_type=jnp.float32)
    e0 = jax.lax.dot_general(qs, k0.astype(jnp.bfloat16),
                             (((1,), (1,)), ((), ())),
                             preferred_element_type=jnp.float32)
    cp1.wait()
    s1 = s1_ref[...].reshape(HB * N_S, HID).astype(jnp.bfloat16)
    k1 = jnp.dot(s1, wk, preferred_element_type=jnp.float32)
    v1 = jnp.dot(s1, wv, preferred_element_type=jnp.float32)
    e1 = jax.lax.dot_general(qs, k1.astype(jnp.bfloat16),
                             (((1,), (1,)), ((), ())),
                             preferred_element_type=jnp.float32)

    e = jnp.concatenate([e0, e1], axis=1)
    e = jnp.where(mask, e, -1e30)
    m = jnp.max(e, axis=1, keepdims=True)
    ex = jnp.exp(e - m)
    a = ex / jnp.sum(ex, axis=1, keepdims=True)

    attn = (jnp.dot(a[:, :HC].astype(jnp.bfloat16), v0.astype(jnp.bfloat16),
                    preferred_element_type=jnp.float32)
            + jnp.dot(a[:, HC:].astype(jnp.bfloat16), v1.astype(jnp.bfloat16),
                      preferred_element_type=jnp.float32))
    h = jnp.where(has_vis, attn, ph)
    ho = (jnp.dot(h.astype(jnp.bfloat16),
                  fcw_ref[...].astype(jnp.bfloat16),
                  preferred_element_type=jnp.float32)
          + fcb_ref[...].reshape(1, HID))
    h_out_ref[...] = ho.reshape(B, N_P, HID)

    alpha = a[:, :N_S]
    for g in range(1, B):
        alpha = alpha + a[:, g * N_S:(g + 1) * N_S]
    alpha = jnp.where(has_vis, alpha, 0.0)
    alpha_ref[...] = alpha.reshape(B, N_P, N_S)


def kernel(obs, p_hidden, s_hidden, batch_size, Wq, Wk, Wv, fc_W, fc_b):
    obs_t = jnp.transpose(obs, (0, 2, 1))

    vmem = pl.BlockSpec(memory_space=pltpu.MemorySpace.VMEM)
    h_out, alpha = pl.pallas_call(
        _body,
        in_specs=[
            vmem, vmem,
            pl.BlockSpec(memory_space=pl.ANY),
            vmem, vmem, vmem, vmem, vmem,
        ],
        out_specs=[vmem, vmem],
        out_shape=[
            jax.ShapeDtypeStruct((B, N_P, HID), jnp.float32),
            jax.ShapeDtypeStruct((B, N_P, N_S), jnp.float32),
        ],
        scratch_shapes=[
            pltpu.VMEM((HB, N_S, HID), jnp.float32),
            pltpu.VMEM((HB, N_S, HID), jnp.float32),
            pltpu.SemaphoreType.DMA,
            pltpu.SemaphoreType.DMA,
        ],
    )(obs_t, p_hidden, s_hidden, Wq, Wk, Wv, fc_W, fc_b)
    return h_out, alpha


# speedup vs baseline: 1.1231x; 1.1231x over previous
import jax
import jax.numpy as jnp
from jax.experimental import pallas as pl

N_P = 8
N_S = 64
N_A = N_P + N_S
COMM_RANGE = 0.3
HID = 256
BG = 32

_EXACT = jax.lax.Precision.HIGHEST


def _iota2(shape, dim):
    return jax.lax.broadcasted_iota(jnp.int32, shape, dim)


def _body(obs_ref, ph_ref, s_ref, wq_ref, wk_ref, wv_ref, fcw_ref, fcb_ref,
          h_out_ref, alpha_ref):
    ph = ph_ref[...]
    s = s_ref[...].reshape(BG * N_S, HID)
    R = ph.shape[0]
    C = s.shape[0]

    ob = obs_ref[...]
    pxl = ob[:, 0:1, :N_P].reshape(BG, N_P)
    pyl = ob[:, 1:2, :N_P].reshape(BG, N_P)
    sxl = ob[:, 0:1, N_P:].reshape(BG, N_S)
    syl = ob[:, 1:2, N_P:].reshape(BG, N_S)
    xy = jnp.concatenate([pxl, pyl, sxl, syl], axis=1)
    eb = (_iota2((R, BG), 0) // N_P == _iota2((R, BG), 1)).astype(jnp.float32)
    xyr = jnp.dot(eb, xy, precision=_EXACT)
    sx = xyr[:, 2 * N_P:2 * N_P + N_S]
    sy = xyr[:, 2 * N_P + N_S:]
    lane_eq = _iota2((R, N_P), 1) == _iota2((R, N_P), 0) % N_P
    px = jnp.sum(jnp.where(lane_eq, xyr[:, :N_P], 0.0),
                 axis=1, keepdims=True)
    py = jnp.sum(jnp.where(lane_eq, xyr[:, N_P:2 * N_P], 0.0),
                 axis=1, keepdims=True)

    near2 = jnp.maximum(jnp.abs(px - sx), jnp.abs(py - sy)) <= COMM_RANGE
    has_vis = jnp.any(near2, axis=1, keepdims=True)
    near2f = jnp.where(near2, 1.0, 0.0)
    mask = ((jnp.concatenate([near2f] * BG, axis=1) > 0.5) &
            (_iota2((R, C), 0) // N_P == _iota2((R, C), 1) // N_S))

    sb = s.astype(jnp.bfloat16)
    q = jnp.dot(ph.astype(jnp.bfloat16), wq_ref[...].astype(jnp.bfloat16),
                preferred_element_type=jnp.float32)
    k = jnp.dot(sb, wk_ref[...].astype(jnp.bfloat16),
                preferred_element_type=jnp.float32)
    v = jnp.dot(sb, wv_ref[...].astype(jnp.bfloat16),
                preferred_element_type=jnp.float32)

    e = jax.lax.dot_general((q * (1.0 / 16.0)).astype(jnp.bfloat16),
                            k.astype(jnp.bfloat16),
                            (((1,), (1,)), ((), ())),
                            preferred_element_type=jnp.float32)

    e = jnp.where(mask, e, -1e30)
    m = jnp.max(e, axis=1, keepdims=True)
    ex = jnp.exp(e - m)
    a = ex / jnp.sum(ex, axis=1, keepdims=True)

    attn = jnp.dot(a.astype(jnp.bfloat16), v.astype(jnp.bfloat16),
                   preferred_element_type=jnp.float32)
    h = jnp.where(has_vis, attn, ph)
    ho = (jnp.dot(h.astype(jnp.bfloat16),
                  fcw_ref[...].astype(jnp.bfloat16),
                  preferred_element_type=jnp.float32)
          + fcb_ref[...].reshape(1, HID))
    h_out_ref[...] = ho.reshape(BG, N_P, HID)

    alpha = a[:, :N_S]
    for g in range(1, BG):
        alpha = alpha + a[:, g * N_S:(g + 1) * N_S]
    alpha = jnp.where(has_vis, alpha, 0.0)
    alpha_ref[...] = alpha.reshape(BG, N_P, N_S)


def kernel(obs, p_hidden, s_hidden, batch_size, Wq, Wk, Wv, fc_W, fc_b):
    B = p_hidden.shape[0] // N_P
    G = B // BG
    obs_t = jnp.transpose(obs, (0, 2, 1))

    const2d = pl.BlockSpec((HID, HID), lambda g: (0, 0))
    h_out, alpha = pl.pallas_call(
        _body,
        grid=(G,),
        in_specs=[
            pl.BlockSpec((BG, 8, N_A), lambda g: (g, 0, 0)),
            pl.BlockSpec((BG * N_P, HID), lambda g: (g, 0)),
            pl.BlockSpec((BG, N_S, HID), lambda g: (g, 0, 0)),
            const2d, const2d, const2d, const2d,
            pl.BlockSpec((HID,), lambda g: (0,)),
        ],
        out_specs=[
            pl.BlockSpec((BG, N_P, HID), lambda g: (g, 0, 0)),
            pl.BlockSpec((BG, N_P, N_S), lambda g: (g, 0, 0)),
        ],
        out_shape=[
            jax.ShapeDtypeStruct((B, N_P, HID), jnp.float32),
            jax.ShapeDtypeStruct((B, N_P, N_S), jnp.float32),
        ],
    )(obs_t, p_hidden, s_hidden, Wq, Wk, Wv, fc_W, fc_b)
    return h_out, alpha
